# baseline (speedup 1.0000x reference)
import functools

import jax
import jax.numpy as jnp
from jax import lax
from jax.experimental import pallas as pl
from jax.experimental.pallas import tpu as pltpu
from jax.experimental.pallas import tpu_sc as plsc

N_NODES = 10000
F_DIM = 128

NC = 2
NS = 16
NW = NC * NS

CHUNK = 128
IDXBLK = 4
ROWS_PER_TILE = 640
WB_ROWS = 64
ACC_ROWS = NS * ROWS_PER_TILE


def _sc_body(src_hbm, dst_hbm, h_hbm, out_hbm,
             sblk0, sblk1, dblk0, dblk1, rows0, rows1,
             zbuf, wbuf, acc, gsem0, gsem1, isem0, isem1):
    c = lax.axis_index("c")
    s = lax.axis_index("s")
    wid = s * NC + c
    n_blocks = src_hbm.shape[0] // NW // IDXBLK
    blk_base = wid * n_blocks * IDXBLK
    sblk = (sblk0, sblk1)
    dblk = (dblk0, dblk1)
    rows = (rows0, rows1)
    gsem = (gsem0, gsem1)
    isem = (isem0, isem1)

    for i in range(16):
        for j in range(8):
            zbuf[i, pl.ds(j * 16, 16)] = jnp.zeros((16,), jnp.float32)
    r0 = s * ROWS_PER_TILE

    def zero_step(k, carry):
        pltpu.sync_copy(zbuf, acc.at[pl.ds(r0 + k * 16, 16)])
        return carry

    lax.fori_loop(0, ROWS_PER_TILE // 16, zero_step, 0)
    plsc.subcore_barrier()

    def load_idx(p, m):
        off = blk_base + m * IDXBLK
        pltpu.async_copy(src_hbm.at[pl.ds(off, IDXBLK)], sblk[p], isem[p])
        pltpu.async_copy(dst_hbm.at[pl.ds(off, IDXBLK)], dblk[p], isem[p])

    def wait_idx(p):
        pltpu.make_async_copy(src_hbm.at[pl.ds(blk_base, IDXBLK)],
                              sblk[p], isem[p]).wait()
        pltpu.make_async_copy(src_hbm.at[pl.ds(blk_base, IDXBLK)],
                              dblk[p], isem[p]).wait()

    def issue_gather(b, sidx_ref):
        pltpu.async_copy(h_hbm.at[sidx_ref], rows[b], gsem[b])

    def wait_gather(b):
        pltpu.make_async_copy(h_hbm.at[sblk[0].at[0]], rows[b],
                              gsem[b]).wait()

    def block_body(p, m, prefetch, last):
        for q in range(IDXBLK):
            b = q % 2
            wait_gather(b)
            pltpu.sync_copy(rows[b], acc.at[dblk[p].at[q]], add=True)
            qq = q + 2
            if qq < IDXBLK:
                issue_gather(b, sblk[p].at[qq])
            elif not last:
                if q == IDXBLK - 2:
                    wait_idx(1 - p)
                issue_gather(b, sblk[1 - p].at[qq - IDXBLK])
        if prefetch:
            load_idx(p, m + 2)

    load_idx(0, 0)
    load_idx(1, 1)
    wait_idx(0)
    issue_gather(0, sblk[0].at[0])
    issue_gather(1, sblk[0].at[1])

    def edge_step(k, carry):
        block_body(0, 2 * k, True, False)
        block_body(1, 2 * k + 1, True, False)
        return carry

    lax.fori_loop(0, n_blocks // 2 - 1, edge_step, 0)
    block_body(0, n_blocks - 2, False, False)
    block_body(1, n_blocks - 1, False, True)
    plsc.subcore_barrier()

    def wb_step(k, carry):
        rr = r0 + k * WB_ROWS
        pltpu.sync_copy(acc.at[pl.ds(rr, WB_ROWS)], wbuf)
        pltpu.sync_copy(wbuf, out_hbm.at[c, pl.ds(rr, WB_ROWS)])
        return carry

    lax.fori_loop(0, ROWS_PER_TILE // WB_ROWS, wb_step, 0)


def _sc_segment_sum(src, dst, h):
    mesh = plsc.VectorSubcoreMesh(core_axis_name="c", subcore_axis_name="s")
    fn = pl.kernel(
        _sc_body,
        out_type=jax.ShapeDtypeStruct((NC, ACC_ROWS, F_DIM), jnp.float32),
        mesh=mesh,
        scratch_types=[
            pltpu.VMEM((IDXBLK, CHUNK), jnp.int32),
            pltpu.VMEM((IDXBLK, CHUNK), jnp.int32),
            pltpu.VMEM((IDXBLK, CHUNK), jnp.int32),
            pltpu.VMEM((IDXBLK, CHUNK), jnp.int32),
            pltpu.VMEM((CHUNK, F_DIM), jnp.float32),
            pltpu.VMEM((CHUNK, F_DIM), jnp.float32),
            pltpu.VMEM((16, F_DIM), jnp.float32),
            pltpu.VMEM((WB_ROWS, F_DIM), jnp.float32),
            pltpu.VMEM_SHARED((ACC_ROWS, F_DIM), jnp.float32),
            pltpu.SemaphoreType.DMA,
            pltpu.SemaphoreType.DMA,
            pltpu.SemaphoreType.DMA,
            pltpu.SemaphoreType.DMA,
        ],
    )
    return fn(src, dst, h)


def _tc_fuse_body(x_ref, w_ref, v_ref, p0_ref, p1_ref, o_ref):
    agg = p0_ref[...] + p1_ref[...]
    o_ref[...] = jnp.maximum(
        jnp.dot(x_ref[...], w_ref[...], preferred_element_type=jnp.float32)
        + jnp.dot(agg, v_ref[...], preferred_element_type=jnp.float32),
        0.0,
    )


def _tc_fuse(x, W, V, p0, p1):
    blk = 400
    grid = (N_NODES // blk,)
    return pl.pallas_call(
        _tc_fuse_body,
        grid=grid,
        in_specs=[
            pl.BlockSpec((blk, F_DIM), lambda i: (i, 0)),
            pl.BlockSpec((F_DIM, F_DIM), lambda i: (0, 0)),
            pl.BlockSpec((F_DIM, F_DIM), lambda i: (0, 0)),
            pl.BlockSpec((blk, F_DIM), lambda i: (i, 0)),
            pl.BlockSpec((blk, F_DIM), lambda i: (i, 0)),
        ],
        out_specs=pl.BlockSpec((blk, F_DIM), lambda i: (i, 0)),
        out_shape=jax.ShapeDtypeStruct((N_NODES, F_DIM), jnp.float32),
    )(x, W, V, p0, p1)


def kernel(x, edge_index, h, W, V, alpha):
    ei = edge_index.astype(jnp.int32)
    src = ei[:, 0, :].reshape(-1)
    dst = ei[:, 1, :].reshape(-1)
    total = src.shape[0]
    align = NW * 2 * IDXBLK * CHUNK
    per_w = (-(-total // align) * align) // NW
    pad = NW * per_w - total
    if pad:
        src = jnp.concatenate([src, jnp.zeros((pad,), jnp.int32)])
        dst = jnp.concatenate([dst, jnp.full((pad,), N_NODES, jnp.int32)])
    src = src.reshape(-1, CHUNK)
    dst = dst.reshape(-1, CHUNK)
    partials = _sc_segment_sum(src, dst, h)
    return _tc_fuse(x, W, V, partials[0, :N_NODES], partials[1, :N_NODES])
